# baseline (speedup 1.0000x reference)
import functools

import jax
import jax.numpy as jnp
from jax import lax
from jax.experimental import pallas as pl
from jax.experimental.pallas import tpu as pltpu
from jax.experimental.pallas import tpu_sc as plsc

VOCAB = 100000
EMB = 128
B = 4096
L = 50
LP = 56

_N = B * LP
_NC = 2
_NS = 16
_NW = _NC * _NS
_PER_W = _N // _NW
_C = 128
_NCHUNK = _PER_W // _C


def _make_kernel():
    mesh = plsc.VectorSubcoreMesh(core_axis_name="c", subcore_axis_name="s")

    @functools.partial(
        pl.kernel,
        mesh=mesh,
        out_type=jax.ShapeDtypeStruct((_N, EMB), jnp.float32),
        scratch_types=[
            pltpu.VMEM((_NCHUNK, _C), jnp.int32),
            pltpu.VMEM((_C, EMB), jnp.float32),
            pltpu.VMEM((_C, EMB), jnp.float32),
            pltpu.SemaphoreType.DMA,
            pltpu.SemaphoreType.DMA,
        ],
    )
    def k(idx_hbm, table_hbm, out_hbm, idx_v, rows0, rows1, sem0, sem1):
        wid = lax.axis_index("s") * _NC + lax.axis_index("c")
        base = wid * _PER_W
        pltpu.sync_copy(idx_hbm.at[wid], idx_v)

        def start_gather(j, rows, sem):
            pltpu.async_copy(table_hbm.at[idx_v.at[j]], rows, sem)

        def wait_gather(j, rows, sem):
            pltpu.make_async_copy(table_hbm.at[idx_v.at[j]], rows, sem).wait()

        def write(j, rows):
            pltpu.sync_copy(rows, out_hbm.at[pl.ds(base + j * _C, _C)])

        start_gather(0, rows0, sem0)

        def body(g, _):
            j = 2 * g
            wait_gather(j, rows0, sem0)
            start_gather(j + 1, rows1, sem1)
            write(j, rows0)
            wait_gather(j + 1, rows1, sem1)
            start_gather(j + 2, rows0, sem0)
            write(j + 1, rows1)
            return 0

        lax.fori_loop(0, _NCHUNK // 2 - 1, body, 0)

        j = _NCHUNK - 2
        wait_gather(j, rows0, sem0)
        start_gather(j + 1, rows1, sem1)
        write(j, rows0)
        wait_gather(j + 1, rows1, sem1)
        write(j + 1, rows1)

    return k


_gather_kernel = _make_kernel()


@jax.jit
def kernel(x, weight):
    idx = jnp.pad(x.astype(jnp.int32), ((0, 0), (0, LP - L)))
    idx = idx.reshape(_NW, _NCHUNK, _C)
    out = _gather_kernel(idx, weight)
    return out.reshape(B, LP, EMB)[:, :L, :]


# speedup vs baseline: 11.3879x; 11.3879x over previous
import functools

import jax
import jax.numpy as jnp
from jax import lax
from jax.experimental import pallas as pl
from jax.experimental.pallas import tpu as pltpu
from jax.experimental.pallas import tpu_sc as plsc

VOCAB = 100000
EMB = 128
B = 4096
L = 50

_NC = 2
_NS = 16
_NW = _NC * _NS
_BW = B // _NW


def _make_kernel():
    mesh = plsc.VectorSubcoreMesh(core_axis_name="c", subcore_axis_name="s")

    @functools.partial(
        pl.kernel,
        mesh=mesh,
        out_type=jax.ShapeDtypeStruct((L, B, EMB), jnp.float32),
        scratch_types=[
            pltpu.VMEM((L, _BW), jnp.int32),
            pltpu.VMEM((_BW, EMB), jnp.float32),
            pltpu.VMEM((_BW, EMB), jnp.float32),
            pltpu.SemaphoreType.DMA,
            pltpu.SemaphoreType.DMA,
        ],
    )
    def k(idx_hbm, table_hbm, out_hbm, idx_v, rows0, rows1, sem0, sem1):
        wid = lax.axis_index("s") * _NC + lax.axis_index("c")
        base = wid * _BW
        pltpu.sync_copy(idx_hbm.at[wid], idx_v)

        def start_gather(j, rows, sem):
            pltpu.async_copy(table_hbm.at[idx_v.at[j]], rows, sem)

        def wait_gather(j, rows, sem):
            pltpu.make_async_copy(table_hbm.at[idx_v.at[j]], rows, sem).wait()

        def write(j, rows):
            pltpu.sync_copy(rows, out_hbm.at[j, pl.ds(base, _BW)])

        start_gather(0, rows0, sem0)

        def body(g, _):
            j = 2 * g
            wait_gather(j, rows0, sem0)
            start_gather(j + 1, rows1, sem1)
            write(j, rows0)
            wait_gather(j + 1, rows1, sem1)
            start_gather(j + 2, rows0, sem0)
            write(j + 1, rows1)
            return 0

        lax.fori_loop(0, L // 2 - 1, body, 0)

        j = L - 2
        wait_gather(j, rows0, sem0)
        start_gather(j + 1, rows1, sem1)
        write(j, rows0)
        wait_gather(j + 1, rows1, sem1)
        write(j + 1, rows1)

    return k


_gather_kernel = _make_kernel()


@jax.jit
def kernel(x, weight):
    idx = x.astype(jnp.int32).T.reshape(L, _NW, _BW).transpose(1, 0, 2)
    out = _gather_kernel(idx, weight)
    return out.transpose(1, 0, 2)


# speedup vs baseline: 14.0464x; 1.2335x over previous
import functools

import jax
import jax.numpy as jnp
from jax import lax
from jax.experimental import pallas as pl
from jax.experimental.pallas import tpu as pltpu
from jax.experimental.pallas import tpu_sc as plsc

VOCAB = 100000
EMB = 128
B = 4096
L = 50

_NC = 2
_NS = 16
_NW = _NC * _NS
_BW = B // _NW
_NSLOT = 4


def _make_kernel():
    mesh = plsc.VectorSubcoreMesh(core_axis_name="c", subcore_axis_name="s")

    @functools.partial(
        pl.kernel,
        mesh=mesh,
        out_type=jax.ShapeDtypeStruct((L, B, EMB), jnp.float32),
        scratch_types=[
            pltpu.VMEM((L, _BW), jnp.int32),
            [pltpu.VMEM((_BW, EMB), jnp.float32)] * _NSLOT,
            [pltpu.SemaphoreType.DMA] * _NSLOT,
            [pltpu.SemaphoreType.DMA] * _NSLOT,
        ],
    )
    def k(idx_hbm, table_hbm, out_hbm, idx_v, rows, gsem, wsem):
        wid = lax.axis_index("s") * _NC + lax.axis_index("c")
        base = wid * _BW
        pltpu.sync_copy(idx_hbm.at[wid], idx_v)

        def sg(j, s):
            pltpu.async_copy(table_hbm.at[idx_v.at[j]], rows[s], gsem[s])

        def wg(j, s):
            pltpu.make_async_copy(
                table_hbm.at[idx_v.at[j]], rows[s], gsem[s]).wait()

        def sw(j, s):
            pltpu.async_copy(
                rows[s], out_hbm.at[j, pl.ds(base, _BW)], wsem[s])

        def ww(j, s):
            pltpu.make_async_copy(
                rows[s], out_hbm.at[j, pl.ds(base, _BW)], wsem[s]).wait()

        sg(0, 0)
        sg(1, 1)
        wg(0, 0)
        sg(2, 2)
        sw(0, 0)
        wg(1, 1)
        sg(3, 3)
        sw(1, 1)

        def body(q, _):
            j = 4 * q + 2
            for kk in range(4):
                s = (2 + kk) % _NSLOT
                s2 = kk % _NSLOT
                wg(j + kk, s)
                ww(j + kk - 2, s2)
                sg(j + kk + 2, s2)
                sw(j + kk, s)
            return 0

        lax.fori_loop(0, (L - 6) // 4, body, 0)

        j = L - 4
        wg(j, 2)
        ww(j - 2, 0)
        sg(j + 2, 0)
        sw(j, 2)
        wg(j + 1, 3)
        ww(j - 1, 1)
        sg(j + 3, 1)
        sw(j + 1, 3)
        wg(j + 2, 0)
        ww(j, 2)
        sw(j + 2, 0)
        wg(j + 3, 1)
        ww(j + 1, 3)
        sw(j + 3, 1)
        ww(j + 2, 0)
        ww(j + 3, 1)

    return k


_gather_kernel = _make_kernel()


@jax.jit
def kernel(x, weight):
    idx = x.astype(jnp.int32).T.reshape(L, _NW, _BW).transpose(1, 0, 2)
    out = _gather_kernel(idx, weight)
    return out.transpose(1, 0, 2)


# speedup vs baseline: 14.1856x; 1.0099x over previous
import functools

import jax
import jax.numpy as jnp
from jax import lax
from jax.experimental import pallas as pl
from jax.experimental.pallas import tpu as pltpu
from jax.experimental.pallas import tpu_sc as plsc

VOCAB = 100000
EMB = 128
B = 4096
L = 50

_NC = 2
_NS = 16
_NW = _NC * _NS
_BW = B // _NW
_NSLOT = 6


def _make_kernel():
    mesh = plsc.VectorSubcoreMesh(core_axis_name="c", subcore_axis_name="s")

    @functools.partial(
        pl.kernel,
        mesh=mesh,
        out_type=jax.ShapeDtypeStruct((L, B, EMB), jnp.float32),
        scratch_types=[
            pltpu.VMEM((L, _BW), jnp.int32),
            [pltpu.VMEM((_BW, EMB), jnp.float32)] * _NSLOT,
            [pltpu.SemaphoreType.DMA] * _NSLOT,
            [pltpu.SemaphoreType.DMA] * _NSLOT,
        ],
    )
    def k(idx_hbm, table_hbm, out_hbm, idx_v, rows, gsem, wsem):
        wid = lax.axis_index("s") * _NC + lax.axis_index("c")
        base = wid * _BW
        pltpu.sync_copy(idx_hbm.at[wid], idx_v)

        def sg(j, s):
            pltpu.async_copy(table_hbm.at[idx_v.at[j]], rows[s], gsem[s])

        def wg(j, s):
            pltpu.make_async_copy(
                table_hbm.at[idx_v.at[j]], rows[s], gsem[s]).wait()

        def sw(j, s):
            pltpu.async_copy(
                rows[s], out_hbm.at[j, pl.ds(base, _BW)], wsem[s])

        def ww(j, s):
            pltpu.make_async_copy(
                rows[s], out_hbm.at[j, pl.ds(base, _BW)], wsem[s]).wait()

        sg(0, 0)
        sg(1, 1)
        sg(2, 2)
        wg(0, 0)
        sg(3, 3)
        sw(0, 0)
        wg(1, 1)
        sg(4, 4)
        sw(1, 1)
        wg(2, 2)
        sg(5, 5)
        sw(2, 2)

        def body(q, _):
            j = 6 * q + 3
            for kk in range(6):
                s = (3 + kk) % _NSLOT
                s2 = kk % _NSLOT
                wg(j + kk, s)
                ww(j + kk - 3, s2)
                sg(j + kk + 3, s2)
                sw(j + kk, s)
            return 0

        lax.fori_loop(0, (L - 8) // 6, body, 0)

        j = L - 5
        wg(j, 3)
        ww(j - 3, 0)
        sg(j + 3, 0)
        sw(j, 3)
        wg(j + 1, 4)
        ww(j - 2, 1)
        sg(j + 4, 1)
        sw(j + 1, 4)
        wg(j + 2, 5)
        ww(j - 1, 2)
        sw(j + 2, 5)
        wg(j + 3, 0)
        ww(j, 3)
        sw(j + 3, 0)
        wg(j + 4, 1)
        ww(j + 1, 4)
        sw(j + 4, 1)
        ww(j + 2, 5)
        ww(j + 3, 0)
        ww(j + 4, 1)

    return k


_gather_kernel = _make_kernel()


@jax.jit
def kernel(x, weight):
    idx = x.astype(jnp.int32).T.reshape(L, _NW, _BW).transpose(1, 0, 2)
    out = _gather_kernel(idx, weight)
    return out.transpose(1, 0, 2)
